# baseline (speedup 1.0000x reference)
import jax
import jax.numpy as jnp
from jax.experimental import pallas as pl
from jax.experimental.pallas import tpu as pltpu

_ANCHOR_W = (116.0, 156.0, 373.0)
_ANCHOR_H = (90.0, 198.0, 326.0)
_G = 64
_STRIDE = 16.0
_NC = 11
_ATTRS = 5 + _NC


def _decode_body(x_ref, o_ref):
    v = x_ref[0]

    i = jax.lax.broadcasted_iota(jnp.int32, (1, _G * _G), 1)
    gx = (i % _G).astype(jnp.float32) * _STRIDE
    gy = (i // _G).astype(jnp.float32) * _STRIDE

    for a in range(3):
        s = v[_ATTRS * a:_ATTRS * (a + 1)]
        bx = jax.nn.sigmoid(s[0:1]) * _STRIDE + gx
        by = jax.nn.sigmoid(s[1:2]) * _STRIDE + gy
        bw = jnp.exp(s[2:3]) * _ANCHOR_W[a]
        bh = jnp.exp(s[3:4]) * _ANCHOR_H[a]
        conf = jax.nn.sigmoid(s[4:5])

        logits = s[5:5 + _NC]
        m = jnp.max(logits, axis=0, keepdims=True)
        e = jnp.exp(logits - m)
        z = jnp.sum(e, axis=0, keepdims=True)
        cls = e / z

        res = jnp.concatenate([bx, by, bw, bh, conf, cls], axis=0)
        resT = res.T
        r3 = resT.reshape(_G * _G // 8, 8, _ATTRS)
        packed = jnp.concatenate([r3[:, j, :] for j in range(8)], axis=1)
        o_ref[0, a] = packed


def kernel(x, targets):
    bs = x.shape[0]
    xr = x.reshape(bs, 3 * _ATTRS, _G * _G)
    out = pl.pallas_call(
        _decode_body,
        grid=(bs,),
        in_specs=[pl.BlockSpec((1, 3 * _ATTRS, _G * _G), lambda b: (b, 0, 0))],
        out_specs=pl.BlockSpec((1, 3, _G * _G // 8, 8 * _ATTRS), lambda b: (b, 0, 0, 0)),
        out_shape=jax.ShapeDtypeStruct((bs, 3, _G * _G // 8, 8 * _ATTRS), jnp.float32),
        compiler_params=pltpu.CompilerParams(
            dimension_semantics=("parallel",)
        ),
    )(xr)
    return out.reshape(bs, 3 * _G * _G, _ATTRS)


# speedup vs baseline: 1.7950x; 1.7950x over previous
import jax
import jax.numpy as jnp
from jax.experimental import pallas as pl
from jax.experimental.pallas import tpu as pltpu

_ANCHOR_W = (116.0, 156.0, 373.0)
_ANCHOR_H = (90.0, 198.0, 326.0)
_G = 64
_STRIDE = 16.0
_NC = 11
_ATTRS = 5 + _NC


def _decode_body(x_ref, o_ref):

    i = jax.lax.broadcasted_iota(jnp.int32, (1, _G * _G), 1)
    gx = (i % _G).astype(jnp.float32) * _STRIDE
    gy = (i // _G).astype(jnp.float32) * _STRIDE

    for bb in range(2):
      v = x_ref[bb]
      for a in range(3):
        s = v[_ATTRS * a:_ATTRS * (a + 1)]
        bx = jax.nn.sigmoid(s[0:1]) * _STRIDE + gx
        by = jax.nn.sigmoid(s[1:2]) * _STRIDE + gy
        bw = jnp.exp(s[2:3]) * _ANCHOR_W[a]
        bh = jnp.exp(s[3:4]) * _ANCHOR_H[a]
        conf = jax.nn.sigmoid(s[4:5])

        logits = s[5:5 + _NC]
        m = jnp.max(logits, axis=0, keepdims=True)
        e = jnp.exp(logits - m)
        z = jnp.sum(e, axis=0, keepdims=True)
        cls = e / z

        res = jnp.concatenate([bx, by, bw, bh, conf, cls], axis=0)
        o_ref[bb, a] = res.T


def kernel(x, targets):
    bs = x.shape[0]
    xr = x.reshape(bs, 3 * _ATTRS, _G * _G)
    out = pl.pallas_call(
        _decode_body,
        grid=(bs // 2,),
        in_specs=[pl.BlockSpec((2, 3 * _ATTRS, _G * _G), lambda b: (b, 0, 0))],
        out_specs=pl.BlockSpec((2, 3, _G * _G, _ATTRS), lambda b: (b, 0, 0, 0)),
        out_shape=jax.ShapeDtypeStruct((bs, 3, _G * _G, _ATTRS), jnp.float32),
        compiler_params=pltpu.CompilerParams(
            dimension_semantics=("parallel",)
        ),
    )(xr)
    return out.reshape(bs, 3 * _G * _G, _ATTRS)


# speedup vs baseline: 1.7961x; 1.0006x over previous
import jax
import jax.numpy as jnp
from jax.experimental import pallas as pl
from jax.experimental.pallas import tpu as pltpu

_ANCHOR_W = (116.0, 156.0, 373.0)
_ANCHOR_H = (90.0, 198.0, 326.0)
_G = 64
_STRIDE = 16.0
_NC = 11
_ATTRS = 5 + _NC


def _decode_body(x_ref, o_ref):

    i = jax.lax.broadcasted_iota(jnp.int32, (1, _G * _G), 1)
    gx = (i % _G).astype(jnp.float32) * _STRIDE
    gy = (i // _G).astype(jnp.float32) * _STRIDE

    for bb in range(2):
      v = x_ref[bb]
      for a in range(3):
        s = v[_ATTRS * a:_ATTRS * (a + 1)]
        bx = jax.nn.sigmoid(s[0:1]) * _STRIDE + gx
        by = jax.nn.sigmoid(s[1:2]) * _STRIDE + gy
        bw = jnp.exp(s[2:3]) * _ANCHOR_W[a]
        bh = jnp.exp(s[3:4]) * _ANCHOR_H[a]
        conf = jax.nn.sigmoid(s[4:5])

        logits = s[5:5 + _NC]
        m = jnp.max(logits, axis=0, keepdims=True)
        e = jnp.exp(logits - m)
        z = jnp.sum(e, axis=0, keepdims=True)
        cls = e / z

        res = jnp.concatenate([bx, by, bw, bh, conf, cls], axis=0)
        for k in range(4):
            o_ref[bb, a, 1024 * k:1024 * (k + 1), :] = res[:, 1024 * k:1024 * (k + 1)].T


def kernel(x, targets):
    bs = x.shape[0]
    xr = x.reshape(bs, 3 * _ATTRS, _G * _G)
    out = pl.pallas_call(
        _decode_body,
        grid=(bs // 2,),
        in_specs=[pl.BlockSpec((2, 3 * _ATTRS, _G * _G), lambda b: (b, 0, 0))],
        out_specs=pl.BlockSpec((2, 3, _G * _G, _ATTRS), lambda b: (b, 0, 0, 0)),
        out_shape=jax.ShapeDtypeStruct((bs, 3, _G * _G, _ATTRS), jnp.float32),
        compiler_params=pltpu.CompilerParams(
            dimension_semantics=("parallel",)
        ),
    )(xr)
    return out.reshape(bs, 3 * _G * _G, _ATTRS)
